# baseline (speedup 1.0000x reference)
import jax
import jax.numpy as jnp
from jax import lax
from jax.experimental import pallas as pl
from jax.experimental.pallas import tpu as pltpu
from jax.experimental.pallas import tpu_sc as plsc

NC = 2
NS = 16
LANES = 16
NW = NC * NS


def _sc_segment_matvec(xT, pk, w, zeros_blk):
    D, N = xT.shape
    E = pk.shape[0]
    F = D // NW
    CHUNK = 8000
    NCHUNK = E // CHUNK
    GROUPS = CHUNK // LANES
    assert E % CHUNK == 0 and CHUNK % LANES == 0 and D % NW == 0

    mesh = plsc.VectorSubcoreMesh(core_axis_name="c", subcore_axis_name="s")

    def body(xT_hbm, pk_hbm, w_hbm, zero_hbm, agg_hbm,
             xblk, aggblk, pkb0, pkb1, wb0, wb1, sem0, sem1):
        pkb = (pkb0, pkb1)
        wb = (wb0, wb1)
        c = lax.axis_index("c")
        s = lax.axis_index("s")
        wid = s * NC + c
        ebase = wid * (F * N)
        pltpu.sync_copy(xT_hbm.at[pl.ds(ebase, F * N)], xblk)
        pltpu.sync_copy(zero_hbm, aggblk)

        sems = (sem0, sem1)

        def start(ci, slot):
            base = ci * CHUNK
            return (
                pltpu.async_copy(pk_hbm.at[pl.ds(base, CHUNK)], pkb[slot], sems[slot]),
                pltpu.async_copy(w_hbm.at[pl.ds(base, CHUNK)], wb[slot], sems[slot]),
            )

        for s01 in range(2):
            start(s01, s01)

        def pair(cp, carry):
            for s01 in range(2):
                ci = 2 * cp + s01
                pltpu.make_async_copy(pk_hbm.at[pl.ds(0, CHUNK)], pkb[s01], sems[s01]).wait()
                pltpu.make_async_copy(w_hbm.at[pl.ds(0, CHUNK)], wb[s01], sems[s01]).wait()

                def _grp(gi, c2):
                    off = gi * LANES
                    pv = pkb[s01][pl.ds(off, LANES)]
                    sv = lax.shift_right_logical(pv, 14)
                    dv = lax.bitwise_and(pv, 16383)
                    wv = wb[s01][pl.ds(off, LANES)]
                    for f in range(F):
                        fofs = jnp.full((LANES,), f * N, dtype=jnp.int32)
                        g = plsc.load_gather(xblk, [sv + fofs])
                        plsc.addupdate_scatter(aggblk, [dv + fofs], g * wv)
                    return c2

                lax.fori_loop(0, GROUPS, _grp, 0, unroll=10)

                @pl.when(ci + 2 < NCHUNK)
                def _():
                    base = (ci + 2) * CHUNK
                    pltpu.make_async_copy(pk_hbm.at[pl.ds(base, CHUNK)], pkb[s01], sems[s01]).start()
                    pltpu.make_async_copy(w_hbm.at[pl.ds(base, CHUNK)], wb[s01], sems[s01]).start()
            return carry

        lax.fori_loop(0, NCHUNK // 2, pair, 0)

        pltpu.sync_copy(aggblk, agg_hbm.at[pl.ds(ebase, F * N)])

    run = pl.kernel(
        body,
        out_type=jax.ShapeDtypeStruct((D * N,), jnp.float32),
        mesh=mesh,
        compiler_params=pltpu.CompilerParams(needs_layout_passes=False),
        scratch_types=[
            pltpu.VMEM((F * N,), jnp.float32),
            pltpu.VMEM((F * N,), jnp.float32),
            pltpu.VMEM((CHUNK,), jnp.int32),
            pltpu.VMEM((CHUNK,), jnp.int32),
            pltpu.VMEM((CHUNK,), jnp.float32),
            pltpu.VMEM((CHUNK,), jnp.float32),
            pltpu.SemaphoreType.DMA,
            pltpu.SemaphoreType.DMA,
        ],
    )
    return run(xT.reshape(D * N), pk, w, zeros_blk.reshape(-1)).reshape(D, N)


def _tc_edge_weights(wvec, bvec, c0, c1, c2):
    R = c0.shape[0]

    def body(w_ref, b_ref, c0_ref, c1_ref, c2_ref, o_ref):
        o_ref[...] = jnp.maximum(
            c0_ref[...] * w_ref[0] + c1_ref[...] * w_ref[1]
            + c2_ref[...] * w_ref[2] + b_ref[0], 0.0)

    return pl.pallas_call(
        body,
        out_shape=jax.ShapeDtypeStruct((R, 128), jnp.float32),
        in_specs=[pl.BlockSpec(memory_space=pltpu.SMEM),
                  pl.BlockSpec(memory_space=pltpu.SMEM),
                  pl.BlockSpec((R, 128), lambda: (0, 0)),
                  pl.BlockSpec((R, 128), lambda: (0, 0)),
                  pl.BlockSpec((R, 128), lambda: (0, 0))],
        out_specs=pl.BlockSpec((R, 128), lambda: (0, 0)),
    )(wvec, bvec, c0, c1, c2)


def _tc_fuse_transpose(a, b, c):
    N, D = a.shape

    def body(a_ref, b_ref, c_ref, o_ref):
        o_ref[...] = (a_ref[...] + b_ref[...] + c_ref[...]).T

    return pl.pallas_call(
        body,
        out_shape=jax.ShapeDtypeStruct((D, N), jnp.float32),
        in_specs=[pl.BlockSpec((N, D), lambda: (0, 0))] * 3,
        out_specs=pl.BlockSpec((D, N), lambda: (0, 0)),
    )(a, b, c)


def _tc_layer(aggT, xT, W, b2d):
    D, N = aggT.shape

    def body(W_ref, b_ref, agg_ref, x_ref, o_ref):
        h = lax.dot_general(W_ref[...], agg_ref[...], (((0,), (0,)), ((), ())),
                            precision=lax.Precision.HIGHEST,
                            preferred_element_type=jnp.float32)
        o_ref[...] = jnp.maximum(h + b_ref[...], 0.0) + x_ref[...]

    return pl.pallas_call(
        body,
        out_shape=jax.ShapeDtypeStruct((D, N), jnp.float32),
        in_specs=[pl.BlockSpec((D, D), lambda: (0, 0)),
                  pl.BlockSpec((D, 1), lambda: (0, 0)),
                  pl.BlockSpec((D, N), lambda: (0, 0)),
                  pl.BlockSpec((D, N), lambda: (0, 0))],
        out_specs=pl.BlockSpec((D, N), lambda: (0, 0)),
    )(W, b2d, aggT, xT)


def _tc_final(xT, fcWp, fcbp):
    D, N = xT.shape
    P = fcWp.shape[1]

    def body(w_ref, b_ref, x_ref, o_ref):
        o_ref[...] = lax.dot_general(w_ref[...], x_ref[...], (((0,), (0,)), ((), ())),
                                     precision=lax.Precision.HIGHEST,
                                     preferred_element_type=jnp.float32) + b_ref[...]

    return pl.pallas_call(
        body,
        out_shape=jax.ShapeDtypeStruct((P, N), jnp.float32),
        in_specs=[pl.BlockSpec((D, P), lambda: (0, 0)),
                  pl.BlockSpec((P, 1), lambda: (0, 0)),
                  pl.BlockSpec((D, N), lambda: (0, 0))],
        out_specs=pl.BlockSpec((P, N), lambda: (0, 0)),
    )(fcWp, fcbp, xT)


def kernel(x_struct, x_seq, edgeIndex, edgeAttribute, x_antiberty, token_seq, node_size,
           attr_W, attr_b, W0, b0, W1, b1, W2, b2, W3, b3, fc_W, fc_b):
    N, D = x_struct.shape
    E = edgeIndex.shape[1]
    OUT = fc_W.shape[1]

    src = edgeIndex[0]
    dst = edgeIndex[1]
    pk = jnp.left_shift(src, 14) | dst
    R = E // 128
    c0 = edgeAttribute[:, 0].reshape(R, 128)
    c1 = edgeAttribute[:, 1].reshape(R, 128)
    c2 = edgeAttribute[:, 2].reshape(R, 128)

    atb = _tc_edge_weights(attr_W.ravel(), attr_b, c0, c1, c2).ravel()
    xT = _tc_fuse_transpose(x_struct, x_seq, x_antiberty)

    zeros_blk = jnp.zeros((D // NW, N), jnp.float32)
    for W, b in ((W0, b0), (W1, b1), (W2, b2), (W3, b3)):
        aggT = _sc_segment_matvec(xT, pk, atb, zeros_blk)
        xT = _tc_layer(aggT, xT, W, b.reshape(D, 1))

    P = 8
    fcWp = jnp.zeros((D, P), fc_W.dtype).at[:, :OUT].set(fc_W)
    fcbp = jnp.zeros((P, 1), fc_b.dtype).at[:OUT, 0].set(fc_b)
    outp = _tc_final(xT, fcWp, fcbp)
    return outp[:OUT, :].T


# speedup vs baseline: 1.6372x; 1.6372x over previous
import jax
import jax.numpy as jnp
from jax import lax
from jax.experimental import pallas as pl
from jax.experimental.pallas import tpu as pltpu
from jax.experimental.pallas import tpu_sc as plsc

NC = 2
NS = 16
LANES = 16
NW = NC * NS


def _sc_segment_matvec(xT, pk, w, zeros_blk):
    D, N = xT.shape
    E = pk.shape[0]
    F = D // NW
    CHUNK = 8000
    NCHUNK = E // CHUNK
    GROUPS = CHUNK // LANES
    assert E % CHUNK == 0 and CHUNK % LANES == 0 and D % NW == 0

    mesh = plsc.VectorSubcoreMesh(core_axis_name="c", subcore_axis_name="s")

    def body(xT_hbm, pk_hbm, w_hbm, zero_hbm, agg_hbm,
             xblk, aggblk, pkb0, pkb1, wb0, wb1, sem0, sem1):
        pkb = (pkb0, pkb1)
        wb = (wb0, wb1)
        c = lax.axis_index("c")
        s = lax.axis_index("s")
        wid = s * NC + c
        ebase = wid * (F * N)
        pltpu.sync_copy(xT_hbm.at[pl.ds(ebase, F * N)], xblk)
        pltpu.sync_copy(zero_hbm, aggblk)

        sems = (sem0, sem1)

        def start(ci, slot):
            base = ci * CHUNK
            return (
                pltpu.async_copy(pk_hbm.at[pl.ds(base, CHUNK)], pkb[slot], sems[slot]),
                pltpu.async_copy(w_hbm.at[pl.ds(base, CHUNK)], wb[slot], sems[slot]),
            )

        for s01 in range(2):
            start(s01, s01)

        def pair(cp, carry):
            for s01 in range(2):
                ci = 2 * cp + s01
                pltpu.make_async_copy(pk_hbm.at[pl.ds(0, CHUNK)], pkb[s01], sems[s01]).wait()
                pltpu.make_async_copy(w_hbm.at[pl.ds(0, CHUNK)], wb[s01], sems[s01]).wait()

                def _grp(gi, c2):
                    off = gi * LANES
                    pv = pkb[s01][pl.ds(off, LANES)]
                    sv = lax.shift_right_logical(pv, 14)
                    dv = lax.bitwise_and(pv, 16383)
                    wv = wb[s01][pl.ds(off, LANES)]
                    gs = [plsc.load_gather(xblk.at[pl.ds(f * N, N)], [sv])
                          for f in range(F)]
                    for f in range(F):
                        plsc.addupdate_scatter(aggblk.at[pl.ds(f * N, N)], [dv],
                                               gs[f] * wv)
                    return c2

                lax.fori_loop(0, GROUPS, _grp, 0, unroll=10)

                @pl.when(ci + 2 < NCHUNK)
                def _():
                    base = (ci + 2) * CHUNK
                    pltpu.make_async_copy(pk_hbm.at[pl.ds(base, CHUNK)], pkb[s01], sems[s01]).start()
                    pltpu.make_async_copy(w_hbm.at[pl.ds(base, CHUNK)], wb[s01], sems[s01]).start()
            return carry

        lax.fori_loop(0, NCHUNK // 2, pair, 0)

        pltpu.sync_copy(aggblk, agg_hbm.at[pl.ds(ebase, F * N)])

    run = pl.kernel(
        body,
        out_type=jax.ShapeDtypeStruct((D * N,), jnp.float32),
        mesh=mesh,
        compiler_params=pltpu.CompilerParams(needs_layout_passes=False),
        scratch_types=[
            pltpu.VMEM((F * N,), jnp.float32),
            pltpu.VMEM((F * N,), jnp.float32),
            pltpu.VMEM((CHUNK,), jnp.int32),
            pltpu.VMEM((CHUNK,), jnp.int32),
            pltpu.VMEM((CHUNK,), jnp.float32),
            pltpu.VMEM((CHUNK,), jnp.float32),
            pltpu.SemaphoreType.DMA,
            pltpu.SemaphoreType.DMA,
        ],
    )
    return run(xT.reshape(D * N), pk, w, zeros_blk.reshape(-1)).reshape(D, N)


def _tc_edge_weights(wvec, bvec, c0, c1, c2):
    R = c0.shape[0]

    def body(w_ref, b_ref, c0_ref, c1_ref, c2_ref, o_ref):
        o_ref[...] = jnp.maximum(
            c0_ref[...] * w_ref[0] + c1_ref[...] * w_ref[1]
            + c2_ref[...] * w_ref[2] + b_ref[0], 0.0)

    return pl.pallas_call(
        body,
        out_shape=jax.ShapeDtypeStruct((R, 128), jnp.float32),
        in_specs=[pl.BlockSpec(memory_space=pltpu.SMEM),
                  pl.BlockSpec(memory_space=pltpu.SMEM),
                  pl.BlockSpec((R, 128), lambda: (0, 0)),
                  pl.BlockSpec((R, 128), lambda: (0, 0)),
                  pl.BlockSpec((R, 128), lambda: (0, 0))],
        out_specs=pl.BlockSpec((R, 128), lambda: (0, 0)),
    )(wvec, bvec, c0, c1, c2)


def _tc_fuse_transpose(a, b, c):
    N, D = a.shape

    def body(a_ref, b_ref, c_ref, o_ref):
        o_ref[...] = (a_ref[...] + b_ref[...] + c_ref[...]).T

    return pl.pallas_call(
        body,
        out_shape=jax.ShapeDtypeStruct((D, N), jnp.float32),
        in_specs=[pl.BlockSpec((N, D), lambda: (0, 0))] * 3,
        out_specs=pl.BlockSpec((D, N), lambda: (0, 0)),
    )(a, b, c)


def _tc_layer(aggT, xT, W, b2d):
    D, N = aggT.shape

    def body(W_ref, b_ref, agg_ref, x_ref, o_ref):
        h = lax.dot_general(W_ref[...], agg_ref[...], (((0,), (0,)), ((), ())),
                            precision=lax.Precision.HIGHEST,
                            preferred_element_type=jnp.float32)
        o_ref[...] = jnp.maximum(h + b_ref[...], 0.0) + x_ref[...]

    return pl.pallas_call(
        body,
        out_shape=jax.ShapeDtypeStruct((D, N), jnp.float32),
        in_specs=[pl.BlockSpec((D, D), lambda: (0, 0)),
                  pl.BlockSpec((D, 1), lambda: (0, 0)),
                  pl.BlockSpec((D, N), lambda: (0, 0)),
                  pl.BlockSpec((D, N), lambda: (0, 0))],
        out_specs=pl.BlockSpec((D, N), lambda: (0, 0)),
    )(W, b2d, aggT, xT)


def _tc_final(xT, fcWp, fcbp):
    D, N = xT.shape
    P = fcWp.shape[1]

    def body(w_ref, b_ref, x_ref, o_ref):
        o_ref[...] = lax.dot_general(w_ref[...], x_ref[...], (((0,), (0,)), ((), ())),
                                     precision=lax.Precision.HIGHEST,
                                     preferred_element_type=jnp.float32) + b_ref[...]

    return pl.pallas_call(
        body,
        out_shape=jax.ShapeDtypeStruct((P, N), jnp.float32),
        in_specs=[pl.BlockSpec((D, P), lambda: (0, 0)),
                  pl.BlockSpec((P, 1), lambda: (0, 0)),
                  pl.BlockSpec((D, N), lambda: (0, 0))],
        out_specs=pl.BlockSpec((P, N), lambda: (0, 0)),
    )(fcWp, fcbp, xT)


def kernel(x_struct, x_seq, edgeIndex, edgeAttribute, x_antiberty, token_seq, node_size,
           attr_W, attr_b, W0, b0, W1, b1, W2, b2, W3, b3, fc_W, fc_b):
    N, D = x_struct.shape
    E = edgeIndex.shape[1]
    OUT = fc_W.shape[1]

    src = edgeIndex[0]
    dst = edgeIndex[1]
    pk = jnp.left_shift(src, 14) | dst
    R = E // 128
    c0 = edgeAttribute[:, 0].reshape(R, 128)
    c1 = edgeAttribute[:, 1].reshape(R, 128)
    c2 = edgeAttribute[:, 2].reshape(R, 128)

    atb = _tc_edge_weights(attr_W.ravel(), attr_b, c0, c1, c2).ravel()
    xT = _tc_fuse_transpose(x_struct, x_seq, x_antiberty)

    zeros_blk = jnp.zeros((D // NW, N), jnp.float32)
    for W, b in ((W0, b0), (W1, b1), (W2, b2), (W3, b3)):
        aggT = _sc_segment_matvec(xT, pk, atb, zeros_blk)
        xT = _tc_layer(aggT, xT, W, b.reshape(D, 1))

    P = 8
    fcWp = jnp.zeros((D, P), fc_W.dtype).at[:, :OUT].set(fc_W)
    fcbp = jnp.zeros((P, 1), fc_b.dtype).at[:OUT, 0].set(fc_b)
    outp = _tc_final(xT, fcWp, fcbp)
    return outp[:OUT, :].T


# speedup vs baseline: 1.6538x; 1.0101x over previous
import jax
import jax.numpy as jnp
from jax import lax
from jax.experimental import pallas as pl
from jax.experimental.pallas import tpu as pltpu
from jax.experimental.pallas import tpu_sc as plsc

NC = 2
NS = 16
LANES = 16
NW = NC * NS


def _sc_segment_matvec(xT, pk, w):
    D, N = xT.shape
    E = pk.shape[0]
    F = D // NW
    CHUNK = 8000
    NCHUNK = E // CHUNK
    GROUPS = CHUNK // LANES
    assert E % CHUNK == 0 and CHUNK % LANES == 0 and D % NW == 0

    mesh = plsc.VectorSubcoreMesh(core_axis_name="c", subcore_axis_name="s")

    def body(xT_hbm, pk_hbm, w_hbm, agg_hbm,
             xblk, aggblk, pkb0, pkb1, wb0, wb1, sem0, sem1, xsem):
        pkb = (pkb0, pkb1)
        wb = (wb0, wb1)
        c = lax.axis_index("c")
        s = lax.axis_index("s")
        wid = s * NC + c
        ebase = wid * (F * N)

        sems = (sem0, sem1)

        def start(ci, slot):
            base = ci * CHUNK
            return (
                pltpu.async_copy(pk_hbm.at[pl.ds(base, CHUNK)], pkb[slot], sems[slot]),
                pltpu.async_copy(w_hbm.at[pl.ds(base, CHUNK)], wb[slot], sems[slot]),
            )

        for s01 in range(2):
            start(s01, s01)
        pltpu.async_copy(xT_hbm.at[pl.ds(ebase, F * N)], xblk, xsem)

        def _zero(i, c2):
            aggblk[pl.ds(i * LANES, LANES)] = jnp.zeros((LANES,), jnp.float32)
            return c2

        lax.fori_loop(0, (F * N) // LANES, _zero, 0, unroll=10)
        pltpu.make_async_copy(xT_hbm.at[pl.ds(0, F * N)], xblk, xsem).wait()

        def pair(cp, carry):
            for s01 in range(2):
                ci = 2 * cp + s01
                pltpu.make_async_copy(pk_hbm.at[pl.ds(0, CHUNK)], pkb[s01], sems[s01]).wait()
                pltpu.make_async_copy(w_hbm.at[pl.ds(0, CHUNK)], wb[s01], sems[s01]).wait()

                def _grp(gi, c2):
                    off = gi * LANES
                    pv = pkb[s01][pl.ds(off, LANES)]
                    sv = lax.shift_right_logical(pv, 14)
                    dv = lax.bitwise_and(pv, 16383)
                    wv = wb[s01][pl.ds(off, LANES)]
                    gs = [plsc.load_gather(xblk.at[pl.ds(f * N, N)], [sv])
                          for f in range(F)]
                    for f in range(F):
                        plsc.addupdate_scatter(aggblk.at[pl.ds(f * N, N)], [dv],
                                               gs[f] * wv)
                    return c2

                lax.fori_loop(0, GROUPS, _grp, 0, unroll=20)

                @pl.when(ci + 2 < NCHUNK)
                def _():
                    base = (ci + 2) * CHUNK
                    pltpu.make_async_copy(pk_hbm.at[pl.ds(base, CHUNK)], pkb[s01], sems[s01]).start()
                    pltpu.make_async_copy(w_hbm.at[pl.ds(base, CHUNK)], wb[s01], sems[s01]).start()
            return carry

        lax.fori_loop(0, NCHUNK // 2, pair, 0)

        pltpu.sync_copy(aggblk, agg_hbm.at[pl.ds(ebase, F * N)])

    run = pl.kernel(
        body,
        out_type=jax.ShapeDtypeStruct((D * N,), jnp.float32),
        mesh=mesh,
        compiler_params=pltpu.CompilerParams(needs_layout_passes=False),
        scratch_types=[
            pltpu.VMEM((F * N,), jnp.float32),
            pltpu.VMEM((F * N,), jnp.float32),
            pltpu.VMEM((CHUNK,), jnp.int32),
            pltpu.VMEM((CHUNK,), jnp.int32),
            pltpu.VMEM((CHUNK,), jnp.float32),
            pltpu.VMEM((CHUNK,), jnp.float32),
            pltpu.SemaphoreType.DMA,
            pltpu.SemaphoreType.DMA,
            pltpu.SemaphoreType.DMA,
        ],
    )
    return run(xT.reshape(D * N), pk, w).reshape(D, N)


def _tc_edge_weights(wvec, bvec, c0, c1, c2):
    R = c0.shape[0]

    def body(w_ref, b_ref, c0_ref, c1_ref, c2_ref, o_ref):
        o_ref[...] = jnp.maximum(
            c0_ref[...] * w_ref[0] + c1_ref[...] * w_ref[1]
            + c2_ref[...] * w_ref[2] + b_ref[0], 0.0)

    return pl.pallas_call(
        body,
        out_shape=jax.ShapeDtypeStruct((R, 128), jnp.float32),
        in_specs=[pl.BlockSpec(memory_space=pltpu.SMEM),
                  pl.BlockSpec(memory_space=pltpu.SMEM),
                  pl.BlockSpec((R, 128), lambda: (0, 0)),
                  pl.BlockSpec((R, 128), lambda: (0, 0)),
                  pl.BlockSpec((R, 128), lambda: (0, 0))],
        out_specs=pl.BlockSpec((R, 128), lambda: (0, 0)),
    )(wvec, bvec, c0, c1, c2)


def _tc_fuse_transpose(a, b, c):
    N, D = a.shape

    def body(a_ref, b_ref, c_ref, o_ref):
        o_ref[...] = (a_ref[...] + b_ref[...] + c_ref[...]).T

    return pl.pallas_call(
        body,
        out_shape=jax.ShapeDtypeStruct((D, N), jnp.float32),
        in_specs=[pl.BlockSpec((N, D), lambda: (0, 0))] * 3,
        out_specs=pl.BlockSpec((D, N), lambda: (0, 0)),
    )(a, b, c)


def _tc_layer(aggT, xT, W, b2d):
    D, N = aggT.shape

    def body(W_ref, b_ref, agg_ref, x_ref, o_ref):
        h = lax.dot_general(W_ref[...], agg_ref[...], (((0,), (0,)), ((), ())),
                            precision=lax.Precision.HIGHEST,
                            preferred_element_type=jnp.float32)
        o_ref[...] = jnp.maximum(h + b_ref[...], 0.0) + x_ref[...]

    return pl.pallas_call(
        body,
        out_shape=jax.ShapeDtypeStruct((D, N), jnp.float32),
        in_specs=[pl.BlockSpec((D, D), lambda: (0, 0)),
                  pl.BlockSpec((D, 1), lambda: (0, 0)),
                  pl.BlockSpec((D, N), lambda: (0, 0)),
                  pl.BlockSpec((D, N), lambda: (0, 0))],
        out_specs=pl.BlockSpec((D, N), lambda: (0, 0)),
    )(W, b2d, aggT, xT)


def _tc_final(xT, fcWp, fcbp):
    D, N = xT.shape
    P = fcWp.shape[1]

    def body(w_ref, b_ref, x_ref, o_ref):
        o_ref[...] = lax.dot_general(w_ref[...], x_ref[...], (((0,), (0,)), ((), ())),
                                     precision=lax.Precision.HIGHEST,
                                     preferred_element_type=jnp.float32) + b_ref[...]

    return pl.pallas_call(
        body,
        out_shape=jax.ShapeDtypeStruct((P, N), jnp.float32),
        in_specs=[pl.BlockSpec((D, P), lambda: (0, 0)),
                  pl.BlockSpec((P, 1), lambda: (0, 0)),
                  pl.BlockSpec((D, N), lambda: (0, 0))],
        out_specs=pl.BlockSpec((P, N), lambda: (0, 0)),
    )(fcWp, fcbp, xT)


def kernel(x_struct, x_seq, edgeIndex, edgeAttribute, x_antiberty, token_seq, node_size,
           attr_W, attr_b, W0, b0, W1, b1, W2, b2, W3, b3, fc_W, fc_b):
    N, D = x_struct.shape
    E = edgeIndex.shape[1]
    OUT = fc_W.shape[1]

    src = edgeIndex[0]
    dst = edgeIndex[1]
    pk = jnp.left_shift(src, 14) | dst
    R = E // 128
    c0 = edgeAttribute[:, 0].reshape(R, 128)
    c1 = edgeAttribute[:, 1].reshape(R, 128)
    c2 = edgeAttribute[:, 2].reshape(R, 128)

    atb = _tc_edge_weights(attr_W.ravel(), attr_b, c0, c1, c2).ravel()
    xT = _tc_fuse_transpose(x_struct, x_seq, x_antiberty)

    for W, b in ((W0, b0), (W1, b1), (W2, b2), (W3, b3)):
        aggT = _sc_segment_matvec(xT, pk, atb)
        xT = _tc_layer(aggT, xT, W, b.reshape(D, 1))

    P = 8
    fcWp = jnp.zeros((D, P), fc_W.dtype).at[:, :OUT].set(fc_W)
    fcbp = jnp.zeros((P, 1), fc_b.dtype).at[:OUT, 0].set(fc_b)
    outp = _tc_final(xT, fcWp, fcbp)
    return outp[:OUT, :].T
